# Initial kernel scaffold; baseline (speedup 1.0000x reference)
import jax
import jax.numpy as jnp
from jax.experimental import pallas as pl


def kernel(x, table):
    raise NotImplementedError("write your pallas kernel here")


# speedup vs baseline: 50.3981x; 50.3981x over previous
import math

import jax
import jax.numpy as jnp
from jax.experimental import pallas as pl
from jax.experimental.pallas import tpu as pltpu

_PRIMES = (1, 2654435761, 805459861)

_INPUT_DIM = 3
_NUM_LEVELS = 16
_LEVEL_DIM = 2
_PER_LEVEL_SCALE = 2
_BASE_RESOLUTION = 16
_LOG2_HASHMAP_SIZE = 13

_LANES = 128
_CORNERS = 8


def _i32(v):
    v = int(v) & 0xFFFFFFFF
    return v - (1 << 32) if v >= (1 << 31) else v


def _level_configs():
    log2_scale = math.log2(_PER_LEVEL_SCALE)
    cap = 1 << _LOG2_HASHMAP_SIZE
    cfgs = []
    offset = 0
    for l in range(_NUM_LEVELS):
        scale = (2.0 ** (l * log2_scale)) * _BASE_RESOLUTION - 1.0
        res = int(math.ceil(scale)) + 1
        dense = res ** _INPUT_DIM
        size = min(dense, cap)
        size = ((size + 7) // 8) * 8
        hashed = dense > cap
        cfgs.append(dict(scale=scale, res=res, hashed=hashed,
                         offset=offset, size=size))
        offset += size
    return cfgs, offset


def _pack_table(table, cfgs):
    bits = jax.lax.bitcast_convert_type(table.astype(jnp.bfloat16), jnp.uint16)
    packed = bits[:, 0].astype(jnp.uint32) | (bits[:, 1].astype(jnp.uint32) << 16)
    packed = jax.lax.bitcast_convert_type(packed, jnp.int32)
    parts = []
    for cfg in cfgs:
        off, sz = cfg["offset"], cfg["size"]
        rows = packed[off:off + sz].reshape(sz // _LANES, 1, _LANES)
        parts.append(jnp.broadcast_to(rows, (sz // _LANES, 8, _LANES)))
    return jnp.concatenate(parts, axis=0).reshape(-1, _LANES)


def _make_body(cfgs):
    out_dim = _NUM_LEVELS * _LEVEL_DIM
    row_bases = []
    rb = 0
    for cfg in cfgs:
        row_bases.append(rb)
        rb += cfg["size"] // _LANES

    def body(xt_ref, tbl_ref, out_ref):
        xt = xt_ref[...].astype(jnp.float32)
        xb = [jnp.broadcast_to(xt[d:d + 1, :], (_CORNERS, _LANES))
              for d in range(_INPUT_DIM)]

        si = jax.lax.broadcasted_iota(jnp.int32, (_CORNERS, _LANES), 0)
        masks = [((si >> d) & 1) == 1 for d in range(_INPUT_DIM)]

        for l, cfg in enumerate(cfgs):
            scale = jnp.float32(cfg["scale"])
            res = cfg["res"]
            size = cfg["size"]
            nrows = size // _LANES
            rbase = row_bases[l]

            coords = []
            w = None
            for d in range(_INPUT_DIM):
                pos = xb[d] * scale + jnp.float32(0.5)
                pf = jnp.floor(pos)
                frac = pos - pf
                pg = pf.astype(jnp.int32)
                coords.append(jnp.where(masks[d], pg + 1, pg))
                wd = jnp.where(masks[d], frac, jnp.float32(1.0) - frac)
                w = wd if w is None else w * wd

            if cfg["hashed"]:
                h = coords[0]
                for d in range(1, _INPUT_DIM):
                    h = h ^ (coords[d] * jnp.int32(_i32(_PRIMES[d])))
                idx = h & jnp.int32(size - 1)
            else:
                idx = coords[0]
                stride = 1
                for d in range(1, _INPUT_DIM):
                    stride *= res
                    idx = idx + coords[d] * stride
                idx = jnp.where(idx >= size, idx - size, idx)

            q = idx >> 7
            c = idx & 127

            accs = [jnp.zeros((_CORNERS, _LANES), jnp.int32) for _ in range(4)]
            for j in range(nrows):
                data = tbl_ref[pl.ds((rbase + j) * 8, 8), :]
                g = jnp.take_along_axis(data, c, axis=1)
                accs[j % 4] = jnp.where(q == j, g, accs[j % 4])
            acc = (accs[0] | accs[1]) | (accs[2] | accs[3])

            f0 = pltpu.bitcast(acc << 16, jnp.float32)
            f1 = pltpu.bitcast(acc & jnp.int32(_i32(0xFFFF0000)), jnp.float32)

            r0 = jnp.sum(w * f0, axis=0, keepdims=True)
            r1 = jnp.sum(w * f1, axis=0, keepdims=True)
            out_ref[pl.ds(_LEVEL_DIM * l, 1), :] = r0
            out_ref[pl.ds(_LEVEL_DIM * l + 1, 1), :] = r1

    return body, rb, out_dim


def kernel(x, table):
    cfgs, _total = _level_configs()
    B = x.shape[0]
    body, total_rows, out_dim = _make_body(cfgs)

    tbl = _pack_table(table, cfgs)
    xt = x.T

    out_t = pl.pallas_call(
        body,
        out_shape=jax.ShapeDtypeStruct((out_dim, B), jnp.float32),
        grid_spec=pltpu.PrefetchScalarGridSpec(
            num_scalar_prefetch=0,
            grid=(B // _LANES,),
            in_specs=[
                pl.BlockSpec((_INPUT_DIM, _LANES), lambda i: (0, i)),
                pl.BlockSpec((total_rows * 8, _LANES), lambda i: (0, 0)),
            ],
            out_specs=pl.BlockSpec((out_dim, _LANES), lambda i: (0, i)),
        ),
        compiler_params=pltpu.CompilerParams(
            dimension_semantics=("parallel",),
            vmem_limit_bytes=32 * 1024 * 1024),
    )(xt, tbl)
    return out_t.T
